# Initial kernel scaffold; baseline (speedup 1.0000x reference)
import jax
import jax.numpy as jnp
from jax.experimental import pallas as pl


def kernel(x, W):
    raise NotImplementedError("write your pallas kernel here")


# speedup vs baseline: 3.3257x; 3.3257x over previous
import functools

import jax
import jax.numpy as jnp
from jax import lax
from jax.experimental import pallas as pl
from jax.experimental.pallas import tpu as pltpu
from jax.experimental.pallas import tpu_sc as plsc

VOCAB = 100000
HIDDEN = 32
NUM_CHANNELS = 26
BATCH = 4096
HIST = 20

NTOK = BATCH * HIST * NUM_CHANNELS
NC, NS = 2, 16
NW = NC * NS
PER_W = NTOK // NW
IW = 128
JROWS = 13
CHUNK = JROWS * IW
NCHUNK = PER_W // CHUNK


def _body(x_hbm, tab_hbm, offs_hbm, out_hbm, offs_v, idx_v, rows_v, sem):
    wid = lax.axis_index("s") * NC + lax.axis_index("c")
    pltpu.sync_copy(offs_hbm, offs_v)

    def chunk_body(ci, carry):
        base = pl.multiple_of(wid * PER_W + ci * CHUNK, CHUNK)
        pltpu.sync_copy(x_hbm.at[pl.ds(base, CHUNK)], idx_v)
        for i in range(CHUNK // 16):
            sl = pl.ds(i * 16, 16)
            idx_v[sl] = idx_v[sl] + offs_v[sl]
        copies = [
            pltpu.async_copy(
                tab_hbm.at[idx_v.at[pl.ds(j * IW, IW)]],
                rows_v.at[pl.ds(j * IW, IW)],
                sem,
            )
            for j in range(JROWS)
        ]
        for cp in copies:
            cp.wait()
        pltpu.sync_copy(rows_v, out_hbm.at[pl.ds(base, CHUNK)])
        return carry

    lax.fori_loop(0, NCHUNK, chunk_body, 0)


@jax.jit
def kernel(x, W):
    xr = x.reshape(NTOK)
    table = W.reshape(NUM_CHANNELS * VOCAB, HIDDEN)
    offs = (jnp.arange(CHUNK, dtype=jnp.int32) % NUM_CHANNELS) * VOCAB
    run = pl.kernel(
        _body,
        out_type=jax.ShapeDtypeStruct((NTOK, HIDDEN), jnp.float32),
        mesh=plsc.VectorSubcoreMesh(core_axis_name="c", subcore_axis_name="s"),
        scratch_types=[
            pltpu.VMEM((CHUNK,), jnp.int32),
            pltpu.VMEM((CHUNK,), jnp.int32),
            pltpu.VMEM((CHUNK, HIDDEN), jnp.float32),
            pltpu.SemaphoreType.DMA,
        ],
        compiler_params=pltpu.CompilerParams(use_tc_tiling_on_sc=False),
    )
    out = run(xr, table, offs)
    return out.reshape(BATCH, HIST * NUM_CHANNELS, HIDDEN)


# speedup vs baseline: 3.3940x; 1.0205x over previous
import jax
import jax.numpy as jnp
from jax import lax
from jax.experimental import pallas as pl
from jax.experimental.pallas import tpu as pltpu
from jax.experimental.pallas import tpu_sc as plsc

VOCAB = 100000
HIDDEN = 32
NUM_CHANNELS = 26
BATCH = 4096
HIST = 20

NTOK = BATCH * HIST * NUM_CHANNELS
NC, NS = 2, 16
NW = NC * NS
PER_W = NTOK // NW
IW = 128
JROWS = 13
CHUNK = JROWS * IW
NCHUNK = PER_W // CHUNK


def _body(x_hbm, tab_hbm, offs_hbm, out_hbm,
          offs_v, idx0, idx1, rows0, rows1, sg0, sg1, so0, so1):
    wid = lax.axis_index("s") * NC + lax.axis_index("c")
    base_w = wid * PER_W
    idx = (idx0, idx1)
    rows = (rows0, rows1)
    sg = (sg0, sg1)
    so = (so0, so1)

    pltpu.sync_copy(offs_hbm, offs_v)

    def load_idx(c, b):
        base = pl.multiple_of(base_w + c * CHUNK, CHUNK)
        pltpu.sync_copy(x_hbm.at[pl.ds(base, CHUNK)], idx[b])

        def add_body(i, carry):
            sl = pl.ds(i * 16, 16)
            idx[b][sl] = idx[b][sl] + offs_v[sl]
            return carry

        lax.fori_loop(0, CHUNK // 16, add_body, 0)

    def fire_gathers(b):
        for j in range(JROWS):
            sl = pl.ds(j * IW, IW)
            pltpu.async_copy(tab_hbm.at[idx[b].at[sl]], rows[b].at[sl], sg[b])

    def wait_gathers(b):
        pltpu.make_async_copy(tab_hbm.at[pl.ds(0, CHUNK)], rows[b], sg[b]).wait()

    def fire_out(c, b):
        base = pl.multiple_of(base_w + c * CHUNK, CHUNK)
        pltpu.async_copy(rows[b], out_hbm.at[pl.ds(base, CHUNK)], so[b])

    def wait_out(c, b):
        base = pl.multiple_of(base_w + c * CHUNK, CHUNK)
        pltpu.make_async_copy(rows[b], out_hbm.at[pl.ds(base, CHUNK)], so[b]).wait()

    load_idx(0, 0)
    fire_gathers(0)
    load_idx(1, 1)

    wait_gathers(0)
    fire_gathers(1)
    fire_out(0, 0)
    load_idx(2, 0)

    def sub(g, b):
        wait_gathers(b)
        wait_out(g - 1, 1 - b)
        fire_gathers(1 - b)
        fire_out(g, b)
        load_idx(lax.rem(g + 2, NCHUNK), b)

    def pair(t, carry):
        g = 2 * t + 1
        sub(g, 1)
        sub(g + 1, 0)
        return carry

    lax.fori_loop(0, (NCHUNK - 2) // 2, pair, 0)

    wait_gathers(1)
    wait_out(NCHUNK - 2, 0)
    fire_out(NCHUNK - 1, 1)
    wait_out(NCHUNK - 1, 1)


@jax.jit
def kernel(x, W):
    xr = x.reshape(NTOK)
    table = W.reshape(NUM_CHANNELS * VOCAB, HIDDEN)
    offs = (jnp.arange(CHUNK, dtype=jnp.int32) % NUM_CHANNELS) * VOCAB
    run = pl.kernel(
        _body,
        out_type=jax.ShapeDtypeStruct((NTOK, HIDDEN), jnp.float32),
        mesh=plsc.VectorSubcoreMesh(core_axis_name="c", subcore_axis_name="s"),
        scratch_types=[
            pltpu.VMEM((CHUNK,), jnp.int32),
            pltpu.VMEM((CHUNK,), jnp.int32),
            pltpu.VMEM((CHUNK,), jnp.int32),
            pltpu.VMEM((CHUNK, HIDDEN), jnp.float32),
            pltpu.VMEM((CHUNK, HIDDEN), jnp.float32),
            pltpu.SemaphoreType.DMA,
            pltpu.SemaphoreType.DMA,
            pltpu.SemaphoreType.DMA,
            pltpu.SemaphoreType.DMA,
        ],
        compiler_params=pltpu.CompilerParams(use_tc_tiling_on_sc=False),
    )
    out = run(xr, table, offs)
    return out.reshape(BATCH, HIST * NUM_CHANNELS, HIDDEN)
